# baseline (speedup 1.0000x reference)
import functools

import jax
import jax.numpy as jnp
from jax.experimental import pallas as pl
from jax.experimental.pallas import tpu as pltpu


def _fused_kernel(x_ref, w2_ref, w1_ref, o_ref, xpad_ref, o2pad_ref, *, h, w):
    c_in = x_ref.shape[1]
    hw = h * w
    cmid = w2_ref.shape[1]
    cout = w1_ref.shape[1]
    f32 = jnp.float32
    bf16 = jnp.bfloat16

    @pl.when(pl.program_id(0) == 0)
    def _init():
        xpad_ref[...] = jnp.zeros(xpad_ref.shape, bf16)
        o2pad_ref[...] = jnp.zeros(o2pad_ref.shape, bf16)

    for img in range(x_ref.shape[0]):
        xrT = jnp.transpose(jnp.maximum(x_ref[img], 0.0).astype(bf16))

        xpad_ref[img, 1:h + 1, 1:w + 1, :] = xrT.reshape(h, w, c_in)
        xp = xpad_ref[img]
        xsh = [xp[:, dw:dw + w, :] for dw in range(3)]

        pcat = jnp.concatenate(
            [xsh[dw][dh:dh + h].reshape(hw, c_in)
             for dh in range(3) for dw in range(3)], axis=1)
        acc2 = jnp.dot(pcat, w2_ref[...], preferred_element_type=f32)
        out2 = jnp.maximum(acc2, 0.0).astype(bf16)

        o2pad_ref[img, 0:h, 0:w, :] = out2.reshape(h, w, cmid)
        op = o2pad_ref[img]
        osh = [op[:, dw:dw + w, :] for dw in range(2)]

        def patch(dh, dw):
            return osh[dw][dh:dh + h].reshape(hw, cmid)

        pa = patch(0, 0)
        pb = patch(0, 1)
        pc = patch(1, 0)
        pd = patch(1, 1)

        p00 = jnp.dot(jnp.concatenate([pa, xrT], axis=1),
                      w1_ref[0:2 * cmid], preferred_element_type=f32)
        o_ref[img, 0] = p00.reshape(h, w, cout).astype(o_ref.dtype)
        p01 = jnp.dot(jnp.concatenate([pa, pb], axis=1),
                      w1_ref[2 * cmid:4 * cmid], preferred_element_type=f32)
        o_ref[img, 1] = p01.reshape(h, w, cout).astype(o_ref.dtype)
        p10 = jnp.dot(jnp.concatenate([pa, pc], axis=1),
                      w1_ref[4 * cmid:6 * cmid], preferred_element_type=f32)
        o_ref[img, 2] = p10.reshape(h, w, cout).astype(o_ref.dtype)
        p11 = jnp.dot(jnp.concatenate([pa, pb, pc, pd], axis=1),
                      w1_ref[6 * cmid:10 * cmid], preferred_element_type=f32)
        o_ref[img, 3] = p11.reshape(h, w, cout).astype(o_ref.dtype)


def kernel(x, w_d2, w_d1, w_sc):
    n, cin, h, w = x.shape
    cmid = w_d2.shape[1]
    cout = w_d1.shape[1]
    bf16 = jnp.bfloat16
    npair = 4 if n % 4 == 0 else (2 if n % 2 == 0 else 1)

    x2 = x.reshape(n, cin, h * w)

    w2f = jnp.transpose(w_d2, (2, 3, 0, 1))[::-1, ::-1].astype(bf16)
    w2t = w2f.reshape(9 * cin, cmid)
    w1f = jnp.transpose(w_d1, (2, 3, 0, 1))[::-1, ::-1].astype(bf16)
    wsc = w_sc[:, :, 0, 0].astype(bf16)
    sub_kk = ((1, 1), (1, 0), (1, 2), (0, 1), (2, 1),
              (0, 0), (0, 2), (2, 0), (2, 2))
    taps = [w1f[kh, kw] for kh, kw in sub_kk]
    w1t = jnp.concatenate(
        [taps[0], wsc, taps[1], taps[2], taps[3], taps[4],
         taps[5], taps[6], taps[7], taps[8]], axis=0)

    def run(xs):
        m = xs.shape[0]
        phases = pl.pallas_call(
            functools.partial(_fused_kernel, h=h, w=w),
            out_shape=jax.ShapeDtypeStruct((m, 4, h, w, cout), x.dtype),
            grid=(m // npair,),
            in_specs=[
                pl.BlockSpec((npair, cin, h * w), lambda b: (b, 0, 0)),
                pl.BlockSpec((9 * cin, cmid), lambda b: (0, 0)),
                pl.BlockSpec((10 * cmid, cout), lambda b: (0, 0)),
            ],
            out_specs=pl.BlockSpec((npair, 4, h, w, cout),
                                   lambda b: (b, 0, 0, 0, 0)),
            scratch_shapes=[
                pltpu.VMEM((npair, h + 2, w + 2, cin), bf16),
                pltpu.VMEM((npair, h + 1, w + 1, cmid), bf16),
            ],
            compiler_params=pltpu.CompilerParams(
                dimension_semantics=("arbitrary",),
                vmem_limit_bytes=48 * 2 ** 20,
            ),
        )(xs, w2t, w1t)
        ph = phases.reshape(m, 2, 2, h, w, cout)
        ph = jnp.transpose(ph, (0, 5, 3, 1, 4, 2))
        return ph.reshape(m, cout, 2 * h, 2 * w)

    if n % (2 * npair) == 0:
        half = n // 2
        return jnp.concatenate([run(x2[:half]), run(x2[half:])], axis=0)
    return run(x2)


# speedup vs baseline: 1.3667x; 1.3667x over previous
import functools

import jax
import jax.numpy as jnp
from jax.experimental import pallas as pl
from jax.experimental.pallas import tpu as pltpu


def _fused_kernel(x_ref, w2_ref, w1_ref, o_ref, xpad_ref, o2pad_ref, *, h, w):
    c_in = x_ref.shape[1]
    hw = h * w
    cmid = w2_ref.shape[1]
    cout = w1_ref.shape[1]
    f32 = jnp.float32
    bf16 = jnp.bfloat16

    @pl.when(pl.program_id(0) == 0)
    def _init():
        xpad_ref[...] = jnp.zeros(xpad_ref.shape, bf16)
        o2pad_ref[...] = jnp.zeros(o2pad_ref.shape, bf16)

    for img in range(x_ref.shape[0]):
        xrT = jnp.transpose(jnp.maximum(x_ref[img], 0.0).astype(bf16))

        xpad_ref[img, 1:h + 1, 8:8 + w, :] = xrT.reshape(h, w, c_in)
        xp = xpad_ref[img]
        xsh = [xp[:, dw + 7:dw + 7 + w, :] for dw in range(3)]

        pcat = jnp.concatenate(
            [xsh[dw][dh:dh + h].reshape(hw, c_in)
             for dh in range(3) for dw in range(3)], axis=1)
        acc2 = jnp.dot(pcat, w2_ref[...], preferred_element_type=f32)
        out2 = jnp.maximum(acc2, 0.0).astype(bf16)

        o2pad_ref[img, 0:h, 0:w, :] = out2.reshape(h, w, cmid)
        op = o2pad_ref[img]
        osh = [op[:, dw:dw + w, :] for dw in range(2)]

        def patch(dh, dw):
            return osh[dw][dh:dh + h].reshape(hw, cmid)

        pa = patch(0, 0)
        pb = patch(0, 1)
        pc = patch(1, 0)
        pd = patch(1, 1)

        p00 = jnp.dot(jnp.concatenate([pa, xrT], axis=1),
                      w1_ref[0:2 * cmid], preferred_element_type=f32)
        o_ref[img, 0] = p00.reshape(h, w, cout).astype(o_ref.dtype)
        p01 = jnp.dot(jnp.concatenate([pa, pb], axis=1),
                      w1_ref[2 * cmid:4 * cmid], preferred_element_type=f32)
        o_ref[img, 1] = p01.reshape(h, w, cout).astype(o_ref.dtype)
        p10 = jnp.dot(jnp.concatenate([pa, pc], axis=1),
                      w1_ref[4 * cmid:6 * cmid], preferred_element_type=f32)
        o_ref[img, 2] = p10.reshape(h, w, cout).astype(o_ref.dtype)
        p11 = jnp.dot(jnp.concatenate([pa, pb, pc, pd], axis=1),
                      w1_ref[6 * cmid:10 * cmid], preferred_element_type=f32)
        o_ref[img, 3] = p11.reshape(h, w, cout).astype(o_ref.dtype)


def kernel(x, w_d2, w_d1, w_sc):
    n, cin, h, w = x.shape
    cmid = w_d2.shape[1]
    cout = w_d1.shape[1]
    bf16 = jnp.bfloat16
    npair = 4 if n % 4 == 0 else (2 if n % 2 == 0 else 1)

    x2 = x.reshape(n, cin, h * w)

    w2f = jnp.transpose(w_d2, (2, 3, 0, 1))[::-1, ::-1].astype(bf16)
    w2t = w2f.reshape(9 * cin, cmid)
    w1f = jnp.transpose(w_d1, (2, 3, 0, 1))[::-1, ::-1].astype(bf16)
    wsc = w_sc[:, :, 0, 0].astype(bf16)
    sub_kk = ((1, 1), (1, 0), (1, 2), (0, 1), (2, 1),
              (0, 0), (0, 2), (2, 0), (2, 2))
    taps = [w1f[kh, kw] for kh, kw in sub_kk]
    w1t = jnp.concatenate(
        [taps[0], wsc, taps[1], taps[2], taps[3], taps[4],
         taps[5], taps[6], taps[7], taps[8]], axis=0)

    def run(xs):
        m = xs.shape[0]
        phases = pl.pallas_call(
            functools.partial(_fused_kernel, h=h, w=w),
            out_shape=jax.ShapeDtypeStruct((m, 4, h, w, cout), x.dtype),
            grid=(m // npair,),
            in_specs=[
                pl.BlockSpec((npair, cin, h * w), lambda b: (b, 0, 0)),
                pl.BlockSpec((9 * cin, cmid), lambda b: (0, 0)),
                pl.BlockSpec((10 * cmid, cout), lambda b: (0, 0)),
            ],
            out_specs=pl.BlockSpec((npair, 4, h, w, cout),
                                   lambda b: (b, 0, 0, 0, 0)),
            scratch_shapes=[
                pltpu.VMEM((npair, h + 2, w + 16, cin), bf16),
                pltpu.VMEM((npair, h + 1, w + 1, cmid), bf16),
            ],
            compiler_params=pltpu.CompilerParams(
                dimension_semantics=("arbitrary",),
                vmem_limit_bytes=48 * 2 ** 20,
            ),
        )(xs, w2t, w1t)
        ph = phases.reshape(m, 2, 2, h, w, cout)
        ph = jnp.transpose(ph, (0, 5, 3, 1, 4, 2))
        return ph.reshape(m, cout, 2 * h, 2 * w)

    return run(x2)
